# Initial kernel scaffold; baseline (speedup 1.0000x reference)
import jax
import jax.numpy as jnp
from jax.experimental import pallas as pl


def kernel(x, edge_index, edge_attr, cW1, cb1, cW2, cb2, tW1, tb1, tW2, tb2, Wsrc1, Wdst1, asrc1, adst1, We1, ae1, b1, Wsrc2, Wdst2, asrc2, adst2, We2, ae2, b2, Wsrc3, Wdst3, asrc3, adst3, b3):
    raise NotImplementedError("write your pallas kernel here")


# speedup vs baseline: 14.7402x; 14.7402x over previous
import functools
import jax
import jax.numpy as jnp
from jax import lax
from jax.experimental import pallas as pl
from jax.experimental.pallas import tpu as pltpu
from jax.experimental.pallas import tpu_sc as plsc

N = 50000
NP = 50048
E = 800000
C = 128
QW = 8
NW = 32
CPW = 196
EPW = CPW * C
EP = NW * EPW
R = 2176
GRID = NP // R


def _edge_body(src_h, dst_h, ea_h, as_h, ad_h, hs_h, zq_h, zn_h, par_h,
               out_h, den_h,
               as_v, ad_v, sidx_v, didx_v, ea_v, z_v, rows_v, par_v,
               out_sh, den_sh, sem):
    cid = lax.axis_index("c")
    sid = lax.axis_index("s")

    @pl.when(sid == 0)
    def _zero():
        pltpu.sync_copy(zq_h, out_sh)
        pltpu.sync_copy(zn_h, den_sh)

    pltpu.sync_copy(as_h, as_v)
    pltpu.sync_copy(ad_h, ad_v)
    pltpu.sync_copy(par_h, par_v)
    plsc.subcore_barrier()

    wid = cid * 16 + sid
    base = wid * EPW
    gv = plsc.load_gather(par_v, [jnp.zeros((16,), jnp.int32)])
    wv = plsc.load_gather(par_v, [jnp.ones((16,), jnp.int32)])
    iota = lax.iota(jnp.int32, 16)
    pair = iota // 8
    colq = iota % 8

    def chunk(ci, carry):
        off = base + ci * C
        pltpu.sync_copy(src_h.at[pl.ds(off, C)], sidx_v)
        pltpu.sync_copy(dst_h.at[pl.ds(off, C)], didx_v)
        pltpu.sync_copy(ea_h.at[pl.ds(off, C)], ea_v)
        cp = pltpu.async_copy(hs_h.at[sidx_v], rows_v, sem)
        for j in range(C // 16):
            si = sidx_v[pl.ds(j * 16, 16)]
            di = didx_v[pl.ds(j * 16, 16)]
            a = (plsc.load_gather(as_v, [si]) + plsc.load_gather(ad_v, [di])
                 + ea_v[pl.ds(j * 16, 16)] * wv)
            a = jnp.where(a > 0.0, a, 0.2 * a)
            z_v[pl.ds(j * 16, 16)] = jnp.exp(a - gv)
        cp.wait()
        for j in range(0, C, 2):
            zb = plsc.load_gather(z_v, [j + pair])
            rid = j + pair
            v = plsc.load_gather(rows_v, [rid, colq])
            plsc.store_scatter(rows_v, [rid, colq], v * zb)
        pltpu.sync_copy(z_v, den_sh.at[didx_v], add=True)
        pltpu.sync_copy(rows_v, out_sh.at[didx_v], add=True)
        return carry

    lax.fori_loop(0, CPW, chunk, 0)
    plsc.subcore_barrier()

    @pl.when(sid == 0)
    def _dump():
        pltpu.sync_copy(out_sh, out_h.at[cid])
        pltpu.sync_copy(den_sh, den_h.at[cid])


_edge_pass = pl.kernel(
    _edge_body,
    out_type=(
        jax.ShapeDtypeStruct((2, NP, QW), jnp.float32),
        jax.ShapeDtypeStruct((2, NP), jnp.float32),
    ),
    mesh=plsc.VectorSubcoreMesh(core_axis_name="c", subcore_axis_name="s"),
    compiler_params=pltpu.CompilerParams(needs_layout_passes=False,
                                         use_tc_tiling_on_sc=False),
    scratch_types=[
        pltpu.VMEM((NP,), jnp.float32),
        pltpu.VMEM((NP,), jnp.float32),
        pltpu.VMEM((C,), jnp.int32),
        pltpu.VMEM((C,), jnp.int32),
        pltpu.VMEM((C,), jnp.float32),
        pltpu.VMEM((C,), jnp.float32),
        pltpu.VMEM((C, QW), jnp.float32),
        pltpu.VMEM((16,), jnp.float32),
        pltpu.VMEM_SHARED((NP, QW), jnp.float32),
        pltpu.VMEM_SHARED((NP,), jnp.float32),
        pltpu.SemaphoreType.DMA,
    ],
)


def _enc_body(x_ref, cw1, cb1, cw2, cb2, tw1, tb1, tw2, tb2, o_ref):
    xb = x_ref[...]
    mask = xb[:, 0:1]
    xf = xb[:, 1:4]
    c = jnp.maximum(jnp.dot(xf[:, 0:2], cw1[...],
                            preferred_element_type=jnp.float32) + cb1[...], 0.0)
    c = jnp.dot(c, cw2[...], preferred_element_type=jnp.float32) + cb2[...]
    t = jnp.maximum(jnp.dot(xf[:, 2:3], tw1[...],
                            preferred_element_type=jnp.float32) + tb1[...], 0.0)
    t = jnp.dot(t, tw2[...], preferred_element_type=jnp.float32) + tb2[...]
    o_ref[...] = c * (1.0 - mask) + t * mask


def _encode(x_pad, cW1, cb1, cW2, cb2, tW1, tb1, tW2, tb2):
    full = lambda s: pl.BlockSpec(s, lambda i: (0,) * len(s))
    return pl.pallas_call(
        _enc_body,
        grid=(GRID,),
        in_specs=[pl.BlockSpec((R, 5), lambda i: (i, 0)),
                  full((2, 16)), full((1, 16)), full((16, 32)), full((1, 32)),
                  full((1, 16)), full((1, 16)), full((16, 32)), full((1, 32))],
        out_specs=pl.BlockSpec((R, 32), lambda i: (i, 0)),
        out_shape=jax.ShapeDtypeStruct((NP, 32), jnp.float32),
    )(x_pad, cW1, cb1.reshape(1, -1), cW2, cb2.reshape(1, -1),
      tW1, tb1.reshape(1, -1), tW2, tb2.reshape(1, -1))


def _prep_body(nh, h_ref, ws, wd, asm, adm, *outs):
    hb = h_ref[...]
    hs = jnp.dot(hb, ws[...], preferred_element_type=jnp.float32)
    hd = jnp.dot(hb, wd[...], preferred_element_type=jnp.float32)
    nq = 4 * nh
    for q in range(nq):
        outs[q][...] = hs[:, q * QW:(q + 1) * QW]
    outs[nq][...] = jnp.dot(hs, asm[...], preferred_element_type=jnp.float32)
    outs[nq + 1][...] = jnp.dot(hd, adm[...], preferred_element_type=jnp.float32)


def _prep(h_pad, Ws, Wd, Am_s, Am_d, nh):
    f = h_pad.shape[1]
    fo = Ws.shape[1]
    full = lambda s: pl.BlockSpec(s, lambda i: (0,) * len(s))
    blk = lambda c: pl.BlockSpec((R, c), lambda i: (i, 0))
    nq = 4 * nh
    return pl.pallas_call(
        functools.partial(_prep_body, nh),
        grid=(GRID,),
        in_specs=[blk(f), full((f, fo)), full((f, fo)),
                  full((fo, 8)), full((fo, 8))],
        out_specs=[blk(QW)] * nq + [blk(8), blk(8)],
        out_shape=[jax.ShapeDtypeStruct((NP, QW), jnp.float32)] * nq
        + [jax.ShapeDtypeStruct((NP, 8), jnp.float32)] * 2,
    )(h_pad, Ws, Wd, Am_s, Am_d)


RF = 1088


def _final_body(nh, elu, *refs):
    outs = refs[:4 * nh]
    dens = refs[4 * nh:5 * nh]
    bias = refs[5 * nh]
    o_ref = refs[5 * nh + 1]
    parts = []
    for k in range(nh):
        o = jnp.concatenate([outs[4 * k + q][0] + outs[4 * k + q][1]
                             for q in range(4)], axis=1)
        d = dens[k][0, :, 0] + dens[k][1, :, 0] + 1e-16
        parts.append(o / d[:, None])
    v = jnp.concatenate(parts, axis=1) + bias[...]
    if elu:
        v = jnp.where(v > 0.0, v, jnp.exp(jnp.minimum(v, 0.0)) - 1.0)
    o_ref[...] = v


def _final(outPs, denPs, bias, elu):
    nh = len(denPs)
    full = lambda s: pl.BlockSpec(s, lambda i: (0,) * len(s))
    return pl.pallas_call(
        functools.partial(_final_body, nh, elu),
        grid=(NP // RF,),
        in_specs=[pl.BlockSpec((2, RF, QW), lambda i: (0, i, 0))] * (4 * nh)
        + [pl.BlockSpec((2, RF, 1), lambda i: (0, i, 0))] * nh
        + [full((1, 32 * nh))],
        out_specs=pl.BlockSpec((RF, 32 * nh), lambda i: (i, 0)),
        out_shape=jax.ShapeDtypeStruct((NP, 32 * nh), jnp.float32),
    )(*outPs, *[d.reshape(2, NP, 1) for d in denPs], bias.reshape(1, -1))


def _last_body(o3_ref, d3_ref, x_ref, b3_ref, o_ref):
    o = o3_ref[0, :, 0:1] + o3_ref[1, :, 0:1]
    d = d3_ref[0] + d3_ref[1] + 1e-16
    h = o / d[:, None] + b3_ref[0, 0]
    o_ref[...] = h * x_ref[:, 0:1]


def _last(out3P, den3P, x_pad, b3):
    return pl.pallas_call(
        _last_body,
        grid=(GRID,),
        in_specs=[pl.BlockSpec((2, R, QW), lambda i: (0, i, 0)),
                  pl.BlockSpec((2, R), lambda i: (0, i)),
                  pl.BlockSpec((R, 5), lambda i: (i, 0)),
                  pl.BlockSpec((1, 1), lambda i: (0, 0))],
        out_specs=pl.BlockSpec((R, 1), lambda i: (i, 0)),
        out_shape=jax.ShapeDtypeStruct((NP, 1), jnp.float32),
    )(out3P, den3P, x_pad, b3.reshape(1, 1))


def _att_matrix(att, nh):
    a = jnp.zeros((32 * nh, 8), jnp.float32)
    for k in range(nh):
        a = a.at[k * 32:(k + 1) * 32, k].set(att[k])
    return a


def _gat_layer(h_pad, src_p, dst_p, ea_p, ea_absmax, zq, zn,
               Ws, Wd, att_s, att_d, We, ae, nh, first_q_only=False):
    Am_s = _att_matrix(att_s, nh)
    Am_d = _att_matrix(att_d, nh)
    pr = _prep(h_pad, Ws, Wd, Am_s, Am_d, nh)
    hs_q, asP, adP = pr[:4 * nh], pr[4 * nh], pr[4 * nh + 1]
    if We is not None:
        wcoef = jnp.sum(We[0].reshape(nh, 32) * ae, axis=1)
    else:
        wcoef = jnp.zeros((nh,), jnp.float32)
    amax = (jnp.max(asP[:, :nh], axis=0) + jnp.max(adP[:, :nh], axis=0)
            + ea_absmax * jnp.abs(wcoef))
    G = jnp.where(amax > 0.0, amax, 0.2 * amax)
    outPs, denPs = [], []
    nq = 1 if first_q_only else 4 * nh
    for m in range(nq):
        k = m // 4
        par = jnp.zeros((16,), jnp.float32).at[0].set(G[k]).at[1].set(wcoef[k])
        oP, dP = _edge_pass(src_p, dst_p, ea_p, asP[:, k], adP[:, k],
                            hs_q[m], zq, zn, par)
        outPs.append(oP)
        if m % 4 == 0:
            denPs.append(dP)
    return outPs, denPs


@jax.jit
def _run(x, edge_index, edge_attr, cW1, cb1, cW2, cb2, tW1, tb1, tW2, tb2,
         Wsrc1, Wdst1, asrc1, adst1, We1, ae1, b1,
         Wsrc2, Wdst2, asrc2, adst2, We2, ae2, b2,
         Wsrc3, Wdst3, asrc3, adst3, b3):
    src = edge_index[0]
    dst = edge_index[1]
    ea = edge_attr[:, 0]
    pad_e = EP - E
    src_p = jnp.concatenate([src, jnp.full((pad_e,), N, jnp.int32)])
    dst_p = jnp.concatenate([dst, jnp.full((pad_e,), N, jnp.int32)])
    ea_p = jnp.concatenate([ea, jnp.zeros((pad_e,), jnp.float32)])
    ea_absmax = jnp.max(jnp.abs(ea_p))
    x_pad = jnp.pad(x, ((0, NP - N), (0, 0)))
    zq = jnp.zeros((NP, QW), jnp.float32)
    zn = jnp.zeros((NP,), jnp.float32)

    h0 = _encode(x_pad, cW1, cb1, cW2, cb2, tW1, tb1, tW2, tb2)

    o1, d1 = _gat_layer(h0, src_p, dst_p, ea_p, ea_absmax, zq, zn,
                        Wsrc1, Wdst1, asrc1, adst1, We1, ae1, 3)
    h1 = _final(o1, d1, b1, False)
    o2, d2 = _gat_layer(h1, src_p, dst_p, ea_p, ea_absmax, zq, zn,
                        Wsrc2, Wdst2, asrc2, adst2, We2, ae2, 3)
    h2 = _final(o2, d2, b2, True)

    W3s = jnp.zeros((96, 32), jnp.float32).at[:, 0:1].set(Wsrc3)
    W3d = jnp.zeros((96, 32), jnp.float32).at[:, 0:1].set(Wdst3)
    a3s = jnp.zeros((1, 32), jnp.float32).at[0, 0].set(asrc3[0, 0])
    a3d = jnp.zeros((1, 32), jnp.float32).at[0, 0].set(adst3[0, 0])
    o3, d3 = _gat_layer(h2, src_p, dst_p, ea_p, ea_absmax, zq, zn,
                        W3s, W3d, a3s, a3d, None, None, 1, first_q_only=True)
    res = _last(o3[0], d3[0], x_pad, b3)
    return res[:N, 0]


def kernel(x, edge_index, edge_attr, cW1, cb1, cW2, cb2, tW1, tb1, tW2, tb2,
           Wsrc1, Wdst1, asrc1, adst1, We1, ae1, b1,
           Wsrc2, Wdst2, asrc2, adst2, We2, ae2, b2,
           Wsrc3, Wdst3, asrc3, adst3, b3):
    return _run(x, edge_index, edge_attr, cW1, cb1, cW2, cb2, tW1, tb1, tW2,
                tb2, Wsrc1, Wdst1, asrc1, adst1, We1, ae1, b1,
                Wsrc2, Wdst2, asrc2, adst2, We2, ae2, b2,
                Wsrc3, Wdst3, asrc3, adst3, b3)
